# baseline (speedup 1.0000x reference)
import functools

import jax
import jax.numpy as jnp
from jax import lax
from jax.experimental import pallas as pl
from jax.experimental.pallas import tpu as pltpu
from jax.experimental.pallas import tpu_sc as plsc

_VOCAB = 16
_IDX_CHUNK = 128


def _fold_body(e_ref, w_ref, m_ref):
    m_ref[...] = lax.dot_general(
        e_ref[...], w_ref[...],
        dimension_numbers=(((1,), (1,)), ((), ())),
        preferred_element_type=jnp.float32,
    )


def _fold_tables(embed_table, proj_w):
    return pl.pallas_call(
        _fold_body,
        out_shape=jax.ShapeDtypeStruct((_VOCAB, _VOCAB), jnp.float32),
    )(embed_table, proj_w)


@functools.cache
def _make_gather(n_tokens: int):
    info = plsc.get_sparse_core_info()
    nc, ns = info.num_cores, info.num_subcores
    nw = nc * ns
    tok_per_w = n_tokens // nw
    assert tok_per_w * nw == n_tokens and tok_per_w % _IDX_CHUNK == 0
    chunks = tok_per_w // _IDX_CHUNK
    mesh = plsc.VectorSubcoreMesh(core_axis_name="c", subcore_axis_name="s")

    @functools.partial(
        pl.kernel,
        mesh=mesh,
        compiler_params=pltpu.CompilerParams(use_tc_tiling_on_sc=False),
        out_type=jax.ShapeDtypeStruct(
            (n_tokens // _IDX_CHUNK, _IDX_CHUNK, _VOCAB), jnp.float32),
        scratch_types=[
            pltpu.VMEM((chunks, _IDX_CHUNK), jnp.int32),
            pltpu.VMEM((chunks, _IDX_CHUNK, _VOCAB), jnp.float32),
            pltpu.VMEM_SHARED((_VOCAB, _VOCAB), jnp.float32),
            pltpu.SemaphoreType.DMA,
        ],
    )
    def gather_k(m_hbm, idx_hbm, out_hbm, idx_v, rows_v, m_sp, sem):
        del m_sp
        wid = lax.axis_index("s") * nc + lax.axis_index("c")
        row0 = wid * chunks
        pltpu.sync_copy(idx_hbm.at[pl.ds(row0, chunks)], idx_v)
        copies = [
            pltpu.async_copy(m_hbm.at[idx_v.at[j]], rows_v.at[j], sem)
            for j in range(chunks)
        ]
        for c in copies:
            c.wait()
        pltpu.sync_copy(rows_v, out_hbm.at[pl.ds(row0, chunks)])

    return gather_k


def kernel(input_ids, embed_table, proj_w):
    b, s = input_ids.shape
    n_tokens = b * s
    m = _fold_tables(embed_table, proj_w)
    ids = input_ids.reshape(n_tokens // _IDX_CHUNK, _IDX_CHUNK)
    ids = ids.astype(jnp.int32)
    out = _make_gather(n_tokens)(m, ids)
    return out.reshape(b, s, _VOCAB)


# speedup vs baseline: 2.6930x; 2.6930x over previous
import functools

import jax
import jax.numpy as jnp
from jax import lax
from jax.experimental import pallas as pl
from jax.experimental.pallas import tpu as pltpu
from jax.experimental.pallas import tpu_sc as plsc

_VOCAB = 16


def _fold_body(e_ref, w_ref, m_ref):
    m_ref[...] = lax.dot_general(
        e_ref[...], w_ref[...],
        dimension_numbers=(((1,), (1,)), ((), ())),
        preferred_element_type=jnp.float32,
    )


def _fold_tables(embed_table, proj_w):
    return pl.pallas_call(
        _fold_body,
        out_shape=jax.ShapeDtypeStruct((_VOCAB, _VOCAB), jnp.float32),
    )(embed_table, proj_w)


@functools.cache
def _make_gather(n_tokens: int):
    info = plsc.get_sparse_core_info()
    nc, ns = info.num_cores, info.num_subcores
    nw = nc * ns
    tok_per_w = n_tokens // nw
    assert tok_per_w * nw == n_tokens and tok_per_w % 8 == 0
    mesh = plsc.VectorSubcoreMesh(core_axis_name="c", subcore_axis_name="s")

    @functools.partial(
        pl.kernel,
        mesh=mesh,
        compiler_params=pltpu.CompilerParams(use_tc_tiling_on_sc=False),
        out_type=jax.ShapeDtypeStruct((n_tokens, _VOCAB), jnp.float32),
        scratch_types=[
            pltpu.VMEM((tok_per_w,), jnp.int32),
            pltpu.VMEM((tok_per_w, _VOCAB), jnp.float32),
            pltpu.VMEM_SHARED((_VOCAB, _VOCAB), jnp.float32),
            pltpu.SemaphoreType.DMA,
        ],
    )
    def gather_k(m_hbm, idx_hbm, out_hbm, idx_v, rows_v, m_sp, sem):
        wid = lax.axis_index("s") * nc + lax.axis_index("c")
        base = wid * tok_per_w
        @pl.when(lax.axis_index("s") == 0)
        def _():
            pltpu.sync_copy(m_hbm, m_sp)
        plsc.subcore_barrier()
        pltpu.sync_copy(idx_hbm.at[pl.ds(base, tok_per_w)], idx_v)
        pltpu.async_copy(m_sp.at[idx_v], rows_v, sem).wait()
        pltpu.sync_copy(rows_v, out_hbm.at[pl.ds(base, tok_per_w)])

    return gather_k


def kernel(input_ids, embed_table, proj_w):
    b, s = input_ids.shape
    n_tokens = b * s
    m = _fold_tables(embed_table, proj_w)
    ids = input_ids.reshape(n_tokens).astype(jnp.int32)
    out = _make_gather(n_tokens)(m, ids)
    return out.reshape(b, s, _VOCAB)


# speedup vs baseline: 2.7365x; 1.0162x over previous
import functools

import jax
import jax.numpy as jnp
from jax import lax
from jax.experimental import pallas as pl
from jax.experimental.pallas import tpu as pltpu
from jax.experimental.pallas import tpu_sc as plsc

_VOCAB = 16


def _fold_body(e_ref, w_ref, m_ref):
    m_ref[...] = lax.dot_general(
        e_ref[...], w_ref[...],
        dimension_numbers=(((1,), (1,)), ((), ())),
        preferred_element_type=jnp.float32,
    )


def _fold_tables(embed_table, proj_w):
    return pl.pallas_call(
        _fold_body,
        out_shape=jax.ShapeDtypeStruct((_VOCAB, _VOCAB), jnp.float32),
    )(embed_table, proj_w)


@functools.cache
def _make_gather(n_tokens: int):
    info = plsc.get_sparse_core_info()
    nc, ns = info.num_cores, info.num_subcores
    nw = nc * ns
    tok_per_w = n_tokens // nw
    assert tok_per_w * nw == n_tokens and tok_per_w % 8 == 0
    mesh = plsc.VectorSubcoreMesh(core_axis_name="c", subcore_axis_name="s")

    @functools.partial(
        pl.kernel,
        mesh=mesh,
        compiler_params=pltpu.CompilerParams(use_tc_tiling_on_sc=False),
        out_type=jax.ShapeDtypeStruct((n_tokens, _VOCAB), jnp.float32),
        scratch_types=[
            pltpu.VMEM((tok_per_w,), jnp.int32),
            pltpu.VMEM((tok_per_w, _VOCAB), jnp.float32),
            pltpu.VMEM_SHARED((_VOCAB, _VOCAB), jnp.float32),
            pltpu.SemaphoreType.DMA,
            pltpu.SemaphoreType.DMA,
            pltpu.SemaphoreType.DMA,
        ],
    )
    def gather_k(m_hbm, idx_hbm, out_hbm, idx_v, rows_v, m_sp,
                 sem_i, sem_g, sem_o):
        wid = lax.axis_index("s") * nc + lax.axis_index("c")
        base = wid * tok_per_w
        half = tok_per_w // 2
        idx_cp = pltpu.async_copy(
            idx_hbm.at[pl.ds(base, tok_per_w)], idx_v, sem_i)
        @pl.when(lax.axis_index("s") == 0)
        def _():
            pltpu.sync_copy(m_hbm, m_sp)
        plsc.subcore_barrier()
        idx_cp.wait()
        g0 = pltpu.async_copy(
            m_sp.at[idx_v.at[pl.ds(0, half)]], rows_v.at[pl.ds(0, half)],
            sem_g)
        g1 = pltpu.async_copy(
            m_sp.at[idx_v.at[pl.ds(half, half)]],
            rows_v.at[pl.ds(half, half)], sem_o)
        g0.wait()
        w0 = pltpu.async_copy(
            rows_v.at[pl.ds(0, half)], out_hbm.at[pl.ds(base, half)], sem_g)
        g1.wait()
        w1 = pltpu.async_copy(
            rows_v.at[pl.ds(half, half)],
            out_hbm.at[pl.ds(base + half, half)], sem_o)
        w0.wait()
        w1.wait()

    return gather_k


def kernel(input_ids, embed_table, proj_w):
    b, s = input_ids.shape
    n_tokens = b * s
    m = _fold_tables(embed_table, proj_w)
    ids = input_ids.reshape(n_tokens).astype(jnp.int32)
    out = _make_gather(n_tokens)(m, ids)
    return out.reshape(b, s, _VOCAB)


# speedup vs baseline: 2.7432x; 1.0025x over previous
import functools

import jax
import jax.numpy as jnp
from jax import lax
from jax.experimental import pallas as pl
from jax.experimental.pallas import tpu as pltpu
from jax.experimental.pallas import tpu_sc as plsc

_VOCAB = 16


def _fold_body(e_ref, w_ref, m_ref):
    m_ref[...] = lax.dot_general(
        e_ref[...], w_ref[...],
        dimension_numbers=(((1,), (1,)), ((), ())),
        preferred_element_type=jnp.float32,
    )


def _fold_tables(embed_table, proj_w):
    return pl.pallas_call(
        _fold_body,
        out_shape=jax.ShapeDtypeStruct((_VOCAB, _VOCAB), jnp.float32),
    )(embed_table, proj_w)


@functools.cache
def _make_gather(n_tokens: int):
    info = plsc.get_sparse_core_info()
    nc, ns = info.num_cores, info.num_subcores
    nw = nc * ns
    tok_per_w = n_tokens // nw
    assert tok_per_w * nw == n_tokens and tok_per_w % 8 == 0
    mesh = plsc.VectorSubcoreMesh(core_axis_name="c", subcore_axis_name="s")

    @functools.partial(
        pl.kernel,
        mesh=mesh,
        compiler_params=pltpu.CompilerParams(use_tc_tiling_on_sc=False),
        out_type=jax.ShapeDtypeStruct((n_tokens, _VOCAB), jnp.float32),
        scratch_types=[
            pltpu.VMEM((tok_per_w,), jnp.int32),
            pltpu.VMEM((tok_per_w, _VOCAB), jnp.float32),
            pltpu.VMEM_SHARED((_VOCAB, _VOCAB), jnp.float32),
            pltpu.SemaphoreType.DMA,
            pltpu.SemaphoreType.DMA,
            pltpu.SemaphoreType.DMA,
            pltpu.SemaphoreType.DMA,
            pltpu.SemaphoreType.DMA,
        ],
    )
    def gather_k(m_hbm, idx_hbm, out_hbm, idx_v, rows_v, m_sp,
                 sem_i, sem_g, sem_o, sem_q2, sem_q3):
        wid = lax.axis_index("s") * nc + lax.axis_index("c")
        base = wid * tok_per_w
        half = tok_per_w // 2
        idx_cp = pltpu.async_copy(
            idx_hbm.at[pl.ds(base, tok_per_w)], idx_v, sem_i)
        @pl.when(lax.axis_index("s") == 0)
        def _():
            pltpu.sync_copy(m_hbm, m_sp)
        plsc.subcore_barrier()
        idx_cp.wait()
        nq = 4
        q = tok_per_w // nq
        sems = [sem_g, sem_o, sem_q2, sem_q3]
        gathers = [
            pltpu.async_copy(
                m_sp.at[idx_v.at[pl.ds(i * q, q)]],
                rows_v.at[pl.ds(i * q, q)], sems[i])
            for i in range(nq)
        ]
        writes = []
        for i in range(nq):
            gathers[i].wait()
            writes.append(pltpu.async_copy(
                rows_v.at[pl.ds(i * q, q)],
                out_hbm.at[pl.ds(base + i * q, q)], sems[i]))
        for w in writes:
            w.wait()

    return gather_k


def kernel(input_ids, embed_table, proj_w):
    b, s = input_ids.shape
    n_tokens = b * s
    m = _fold_tables(embed_table, proj_w)
    ids = input_ids.reshape(n_tokens).astype(jnp.int32)
    out = _make_gather(n_tokens)(m, ids)
    return out.reshape(b, s, _VOCAB)
